# baseline (speedup 1.0000x reference)
import jax
import jax.numpy as jnp
from jax import lax
from jax.experimental import pallas as pl
from jax.experimental.pallas import tpu as pltpu
from jax.experimental.pallas import tpu_sc as plsc

_N = 32 * 1000
_NC = 91
_NW = 32
_PER = 1024
_PAD = _NW * _PER - _N
_LIMIT = _N * _NC - 1
_CHUNK = 128
_NCHUNK = _PER // _CHUNK


def _sc_body(cls_hbm, tb_hbm, pred_hbm, out_hbm, cls_v, tb_v, idx_v, sub_v,
             pred_v, acc_v, sem):
    wid = lax.axis_index("s") * 2 + lax.axis_index("c")
    base = wid * _PER

    pltpu.sync_copy(cls_hbm.at[pl.ds(base * 1, _PER)], cls_v)
    pltpu.sync_copy(tb_hbm.at[pl.ds(base * 4, _PER * 4)], tb_v)

    iota = lax.iota(jnp.int32, 16)
    ii4 = iota >> 2
    im4 = iota & 3

    copies = []
    for c in range(_NCHUNK):
        for k in range(_CHUNK // 16):
            i = c * _CHUNK + k * 16
            cls16 = cls_v[pl.ds(i, 16)]
            q = jnp.minimum((base + i + iota) * _NC + cls16, _LIMIT)
            idx_v[c, pl.ds(k * 16, 16)] = q >> 2
            sub_v[pl.ds(i, 16)] = (q & 3) * 4
        copies.append(
            pltpu.async_copy(pred_hbm.at[idx_v.at[c]],
                             pred_v.at[pl.ds(c * _CHUNK, _CHUNK)], sem))
    for cp in copies:
        cp.wait()

    def body(j, carry):
        acc, cnt = carry
        t = tb_v[pl.ds(j * 16, 16)]
        r4 = j * 4 + ii4
        off = plsc.load_gather(sub_v, [r4]) + im4
        p = plsc.load_gather(pred_v, [r4, off])
        c16 = plsc.load_gather(cls_v, [r4])
        d = jnp.abs(t - p)
        l = jnp.where(d < 1.0, 0.5 * d * d, d - 0.5)
        m = c16 > 0
        return acc + jnp.where(m, l, 0.0), cnt + jnp.where(m, 1.0, 0.0)

    zero = jnp.zeros((16,), jnp.float32)
    acc, cnt = lax.fori_loop(0, (_PER * 4) // 16, body, (zero, zero))
    acc_v[pl.ds(0, 16)] = acc
    acc_v[pl.ds(16, 16)] = cnt
    pltpu.sync_copy(acc_v, out_hbm.at[wid])


def _tc_finish(part_ref, out_ref):
    p = part_ref[...]
    total = jnp.sum(p[:, :16])
    count = jnp.sum(p[:, 16:])
    loss = jnp.where(count > 0, total / jnp.maximum(count, 1.0), 0.0)
    out_ref[...] = jnp.reshape(loss, (1, 1))


def kernel(target_boxes, target_class_ids, pred_boxes):
    cls = target_class_ids.reshape(-1).astype(jnp.int32)
    cls = jnp.pad(cls, (0, _PAD))
    tb = jnp.pad(target_boxes.reshape(-1, 4), ((0, _PAD), (0, 0))).reshape(-1)
    pred = pred_boxes.reshape(-1, 16)

    mesh = plsc.VectorSubcoreMesh(core_axis_name="c", subcore_axis_name="s")
    sc = pl.kernel(
        _sc_body, mesh=mesh,
        compiler_params=pltpu.CompilerParams(use_tc_tiling_on_sc=False,
                                             needs_layout_passes=False),
        out_type=jax.ShapeDtypeStruct((_NW, 32), jnp.float32),
        scratch_types=[
            pltpu.VMEM((_PER,), jnp.int32),
            pltpu.VMEM((_PER * 4,), jnp.float32),
            pltpu.VMEM((_NCHUNK, _CHUNK), jnp.int32),
            pltpu.VMEM((_PER,), jnp.int32),
            pltpu.VMEM((_PER, 16), jnp.float32),
            pltpu.VMEM((32,), jnp.float32),
            pltpu.SemaphoreType.DMA,
        ],
    )
    partials = sc(cls, tb, pred)

    out = pl.pallas_call(
        _tc_finish,
        out_shape=jax.ShapeDtypeStruct((1, 1), jnp.float32),
    )(partials)
    return out[0, 0]
